# baseline (speedup 1.0000x reference)
import functools

import jax
import jax.numpy as jnp
from jax import lax
from jax.experimental import pallas as pl
from jax.experimental.pallas import tpu as pltpu
from jax.experimental.pallas import tpu_sc as plsc

N = 10000
E = 160000
B = 64
DIN = 128
H1 = 64
H2 = 128
NH = 4
K = 10
ALPHA = 0.1

NP = 10112
DUMMY = 10008
EP = 170240
EPC = EP // 2
EPT = EPC // 16
KA = 280
STEPS_A = EPT // KA
EPT_B = EP // 16
KB = 304
STEPS_B = EPT_B // KB
RPT = NP // 16

_MESH = plsc.VectorSubcoreMesh(core_axis_name="c", subcore_axis_name="s")
_SC_PARAMS = pltpu.CompilerParams(use_tc_tiling_on_sc=False)
_IOTA = lambda: jax.lax.iota(jnp.int32, 16)
_ZCHUNKS = ((0, 280), (280, 280), (560, RPT - 560))


def _gat_a_body(as_hbm, ad_hbm, srcp_hbm, dstp_hbm, z16_hbm,
                ex_hbm, spart_hbm,
                sidx_v, didx_v, asr_v, adr_v, srow_v, stg_v, ssh, sem):
    c = lax.axis_index("c")
    s = lax.axis_index("s")
    for off, size in _ZCHUNKS:
        pltpu.sync_copy(z16_hbm.at[pl.ds(off, size)],
                        ssh.at[pl.ds(s * RPT + off, size)])
    plsc.subcore_barrier()
    base = c * EPC + s * EPT

    def step(i, carry):
        off = base + i * KA
        pltpu.sync_copy(srcp_hbm.at[pl.ds(off, KA)], sidx_v)
        pltpu.sync_copy(dstp_hbm.at[pl.ds(off, KA)], didx_v)
        pltpu.async_copy(as_hbm.at[sidx_v], asr_v, sem).wait()
        pltpu.async_copy(ad_hbm.at[didx_v], adr_v, sem).wait()

        def grp(g, carry2):
            for k in range(8):
                e = 8 * g + k
                v = asr_v[e] + adr_v[e]
                v = jnp.where(v > 0, v, 0.2 * v)
                v = jnp.exp(v)
                v = jnp.where(_IOTA() < 5, v, 0.0)
                srow_v[e] = v
            return carry2

        lax.fori_loop(0, KA // 8, grp, 0)
        pltpu.sync_copy(srow_v, ssh.at[didx_v], add=True)
        pltpu.sync_copy(srow_v, ex_hbm.at[pl.ds(off, KA)])
        return carry

    lax.fori_loop(0, STEPS_A, step, 0)
    plsc.subcore_barrier()
    for off, size in _ZCHUNKS:
        pltpu.sync_copy(ssh.at[pl.ds(s * RPT + off, size)],
                        stg_v.at[pl.ds(0, size)])
        pltpu.sync_copy(stg_v.at[pl.ds(0, size)],
                        spart_hbm.at[pl.ds(c * NP + s * RPT + off, size)])


_gat_a = pl.kernel(
    _gat_a_body,
    out_type=(jax.ShapeDtypeStruct((EP, 16), jnp.float32),
              jax.ShapeDtypeStruct((2 * NP, 16), jnp.float32)),
    scratch_types=[
        pltpu.VMEM((KA,), jnp.int32),
        pltpu.VMEM((KA,), jnp.int32),
        pltpu.VMEM((KA, 16), jnp.float32),
        pltpu.VMEM((KA, 16), jnp.float32),
        pltpu.VMEM((KA, 16), jnp.float32),
        pltpu.VMEM((RPT, 16), jnp.float32),
        pltpu.VMEM_SHARED((NP, 16), jnp.float32),
        pltpu.SemaphoreType.DMA,
    ],
    mesh=_MESH,
    compiler_params=_SC_PARAMS,
)


def _gat_b_body(hw4_hbm, ext_hbm, srcp_hbm, dstp_hbm, z64_hbm,
                out_hbm,
                sidx_v, didx_v, sidx4_v, rows_v, stg_v, exv_v, acc_sh, sem):
    c = lax.axis_index("c")
    s = lax.axis_index("s")
    base = s * EPT_B
    for q in range(2):
        head_off = (2 * c + q) * NP
        head_off_e = (2 * c + q) * EP
        for off, size in _ZCHUNKS:
            pltpu.sync_copy(z64_hbm.at[pl.ds(off, size)],
                            acc_sh.at[pl.ds(s * RPT + off, size)])
        plsc.subcore_barrier()

        def step(i, carry):
            off = base + i * KB
            pltpu.sync_copy(srcp_hbm.at[pl.ds(off, KB)], sidx_v)
            pltpu.sync_copy(dstp_hbm.at[pl.ds(off, KB)], didx_v)

            def shiftm(m, carry2):
                sidx4_v[pl.ds(16 * m, 16)] = (sidx_v[pl.ds(16 * m, 16)]
                                              + head_off)
                return carry2

            lax.fori_loop(0, KB // 16, shiftm, 0)
            pltpu.async_copy(hw4_hbm.at[sidx4_v], rows_v, sem).wait()
            pltpu.sync_copy(ext_hbm.at[pl.ds(head_off_e + off, KB)], exv_v)

            def grp(g, carry2):
                ex16 = exv_v[pl.ds(16 * g, 16)]
                for k in range(16):
                    e = 16 * g + k
                    bq = jnp.full((16,), ex16[k], jnp.float32)
                    for j in range(4):
                        sl = pl.ds(16 * j, 16)
                        rows_v[e, sl] = rows_v[e, sl] * bq
                return carry2

            lax.fori_loop(0, KB // 16, grp, 0)
            pltpu.sync_copy(rows_v, acc_sh.at[didx_v], add=True)
            return carry

        lax.fori_loop(0, STEPS_B, step, 0)
        plsc.subcore_barrier()
        for off, size in _ZCHUNKS:
            pltpu.sync_copy(acc_sh.at[pl.ds(s * RPT + off, size)],
                            stg_v.at[pl.ds(0, size)])
            pltpu.sync_copy(stg_v.at[pl.ds(0, size)],
                            out_hbm.at[pl.ds(head_off + s * RPT + off, size)])
        plsc.subcore_barrier()


_gat_b = pl.kernel(
    _gat_b_body,
    out_type=jax.ShapeDtypeStruct((4 * NP, H1), jnp.float32),
    scratch_types=[
        pltpu.VMEM((KB,), jnp.int32),
        pltpu.VMEM((KB,), jnp.int32),
        pltpu.VMEM((KB,), jnp.int32),
        pltpu.VMEM((KB, H1), jnp.float32),
        pltpu.VMEM((RPT, H1), jnp.float32),
        pltpu.VMEM((KB,), jnp.float32),
        pltpu.VMEM_SHARED((NP, H1), jnp.float32),
        pltpu.SemaphoreType.DMA,
    ],
    mesh=_MESH,
    compiler_params=_SC_PARAMS,
)


def _appnp_sc_body(hp2_hbm, srcp_hbm, dstp_hbm, z64_hbm, out_hbm,
                   sidx_v, sidx2_v, didx_v, rows_v, stg_v, acc_sh, sem):
    c = lax.axis_index("c")
    s = lax.axis_index("s")
    for off, size in _ZCHUNKS:
        pltpu.sync_copy(z64_hbm.at[pl.ds(off, size)],
                        acc_sh.at[pl.ds(s * RPT + off, size)])
    plsc.subcore_barrier()
    base = s * EPT_B
    cNP = c * NP

    def step(i, carry):
        off = base + i * KB
        pltpu.sync_copy(srcp_hbm.at[pl.ds(off, KB)], sidx_v)
        pltpu.sync_copy(dstp_hbm.at[pl.ds(off, KB)], didx_v)

        def shiftm(m, carry2):
            sidx2_v[pl.ds(16 * m, 16)] = sidx_v[pl.ds(16 * m, 16)] + cNP
            return carry2

        lax.fori_loop(0, KB // 16, shiftm, 0)
        pltpu.async_copy(hp2_hbm.at[sidx2_v], rows_v, sem).wait()
        pltpu.sync_copy(rows_v, acc_sh.at[didx_v], add=True)
        return carry

    lax.fori_loop(0, STEPS_B, step, 0)
    plsc.subcore_barrier()
    for off, size in _ZCHUNKS:
        pltpu.sync_copy(acc_sh.at[pl.ds(s * RPT + off, size)],
                        stg_v.at[pl.ds(0, size)])
        pltpu.sync_copy(stg_v.at[pl.ds(0, size)],
                        out_hbm.at[pl.ds(cNP + s * RPT + off, size)])


_appnp_sc = pl.kernel(
    _appnp_sc_body,
    out_type=jax.ShapeDtypeStruct((2 * NP, H1), jnp.float32),
    scratch_types=[
        pltpu.VMEM((KB,), jnp.int32),
        pltpu.VMEM((KB,), jnp.int32),
        pltpu.VMEM((KB,), jnp.int32),
        pltpu.VMEM((KB, H1), jnp.float32),
        pltpu.VMEM((RPT, H1), jnp.float32),
        pltpu.VMEM_SHARED((NP, H1), jnp.float32),
        pltpu.SemaphoreType.DMA,
    ],
    mesh=_MESH,
    compiler_params=_SC_PARAMS,
)


def _combine_mid_body(out2_ref, x0_ref, dinv_ref, o_ref):
    ssum = jnp.concatenate([out2_ref[0:NP, :], out2_ref[NP:2 * NP, :]], 1)
    dinv = dinv_ref[...]
    h = dinv * ((1.0 - ALPHA) * dinv * ssum + ALPHA * x0_ref[...])
    o_ref[0:NP, :] = h[:, :H1]
    o_ref[NP:2 * NP, :] = h[:, H1:]


def _combine_last_body(out2_ref, x0_ref, dinv_ref, o_ref):
    ssum = jnp.concatenate([out2_ref[0:NP, :], out2_ref[NP:2 * NP, :]], 1)
    o_ref[...] = (1.0 - ALPHA) * dinv_ref[...] * ssum + ALPHA * x0_ref[...]


def _combine(out2, x0p, dinvp, last):
    if last:
        return pl.pallas_call(
            _combine_last_body,
            out_shape=jax.ShapeDtypeStruct((NP, H2), jnp.float32),
        )(out2, x0p, dinvp)
    return pl.pallas_call(
        _combine_mid_body,
        out_shape=jax.ShapeDtypeStruct((2 * NP, H1), jnp.float32),
    )(out2, x0p, dinvp)


def _lin1_body(x_ref, w_ref, b_ref, o_ref):
    o_ref[...] = jnp.dot(x_ref[...], w_ref[...],
                         preferred_element_type=jnp.float32) + b_ref[...]


def _lin1(xcat, w, b):
    return pl.pallas_call(
        _lin1_body,
        out_shape=jax.ShapeDtypeStruct((N, H2), jnp.float32),
    )(xcat, w, b[None, :])


def _bn(x, g, b):
    mu = jnp.mean(x, 0)
    var = jnp.var(x, 0)
    return (x - mu) / jnp.sqrt(var + 1e-5) * g + b


def _pad_rows(a):
    return jnp.pad(a, ((0, NP - a.shape[0]), (0, 0)))


def _gat_sc(h, srcp, dstp, p, z16, z64):
    hw = h @ p['gat_w']
    hwr = hw.reshape(N, NH, H1)
    a_s = jnp.sum(hwr * p['a_src'][None], -1)
    a_d = jnp.sum(hwr * p['a_dst'][None], -1)
    z12 = jnp.zeros((N, 12), jnp.float32)
    as_t = _pad_rows(jnp.concatenate([a_s, z12], 1))
    ad_t = _pad_rows(jnp.concatenate([a_d, z12], 1))
    ex, spart = _gat_a(as_t, ad_t, srcp, dstp, z16)
    scomb = spart[:NP] + spart[NP:]
    ext = ex[:, :NH].T.reshape(-1)
    hw4 = jnp.concatenate(
        [_pad_rows(hw[:, i * H1:(i + 1) * H1]) for i in range(NH)], 0)
    outB = _gat_b(hw4, ext, srcp, dstp, z64)
    s_n = scomb[:N, 0:4] + 1e-16
    osum = sum(outB[i * NP:i * NP + N] / s_n[:, i:i + 1] for i in range(NH))
    return osum, scomb


def _att_layer_sc(x, srcp, dstp, p, z16, z64):
    osum, scomb = _gat_sc(x, srcp, dstp, p, z16, z64)
    h = osum @ p['lin1_w']
    h = _bn(x + h, p['bn1_g'], p['bn1_b'])
    h2 = jax.nn.relu(h @ p['lin2_w'] + p['lin2_b'])
    h2 = h2 @ p['lin3_w'] + p['lin3_b']
    return _bn(h2 + h, p['bn2_g'], p['bn2_b']), scomb


def _appnp(x, deg, srcp, dstp, z64):
    dinv = jnp.where(deg > 0, 1.0 / jnp.sqrt(deg), 0.0)
    dinvp = _pad_rows(dinv[:, None])
    x0p = _pad_rows(x)
    hp = dinvp * x0p
    hp2 = jnp.concatenate([hp[:, :H1], hp[:, H1:]], 0)
    for t in range(K):
        out2 = _appnp_sc(hp2, srcp, dstp, z64)
        hp2 = _combine(out2, x0p, dinvp, last=(t == K - 1))
    return hp2[:N]


def _pool(xg, onehot, p):
    gate = (jax.nn.relu(xg @ p['g1_w'] + p['g1_b']) @ p['g2_w'] + p['g2_b'])[:, 0]
    hh = jax.nn.relu(xg @ p['n1_w'] + p['n1_b']) @ p['n2_w'] + p['n2_b']
    ex = jnp.exp(gate)
    ssum = onehot.T @ ex
    a = (ex / (onehot @ ssum + 1e-16))[:, None]
    return onehot.T @ (a * hh)


def kernel(x, edge_index, batch, Omegas, Phis, Lambdas, J, saved_nodes,
           infected_nodes, size_connected, params):
    n = x.shape[0]
    b = Omegas.shape[0]
    loop = jnp.arange(n, dtype=edge_index.dtype)
    pad = jnp.full((EP - E - N,), DUMMY, edge_index.dtype)
    srcp = jnp.concatenate([edge_index[0], loop, pad])
    dstp = jnp.concatenate([edge_index[1], loop, pad])
    z16 = jnp.zeros((NP, 16), jnp.float32)
    z64 = jnp.zeros((NP, H1), jnp.float32)
    onehot = (batch[:, None] == jnp.arange(b)[None, :]).astype(jnp.float32)

    xcat = jnp.concatenate([x, J, size_connected], 1)
    h = _lin1(xcat, params['lin1_w'], params['lin1_b'])
    deg = None
    for i in (1, 2, 3):
        h, scomb = _att_layer_sc(h, srcp, dstp, params['att%d' % i], z16, z64)
        if deg is None:
            deg = scomb[:N, 4]
    h = _appnp(h, deg, srcp, dstp, z64)

    xg = jnp.concatenate([h, J, saved_nodes, infected_nodes], 1)
    g = jnp.concatenate([_pool(xg, onehot, params['pool%d' % j]) for j in (1, 2, 3)], 1)
    xs = jnp.concatenate([h, onehot @ g, J, saved_nodes, infected_nodes,
                          size_connected, onehot @ Omegas, onehot @ Phis,
                          onehot @ Lambdas], 1)
    s = xs @ params['lin3_w'] + params['lin3_b']
    s = jax.nn.leaky_relu(s, 0.2)
    s = _bn(s, params['bn1_g'], params['bn1_b'])
    s = s @ params['lin4_w'] + params['lin4_b']
    s = jax.nn.leaky_relu(s, 0.2)
    s = _bn(s, params['bn2_g'], params['bn2_b'])
    s = jax.nn.sigmoid(s @ params['lin5_w'] + params['lin5_b'])
    return onehot.T @ s


# speedup vs baseline: 1.4198x; 1.4198x over previous
import functools

import jax
import jax.numpy as jnp
from jax import lax
from jax.experimental import pallas as pl
from jax.experimental.pallas import tpu as pltpu
from jax.experimental.pallas import tpu_sc as plsc

N = 10000
E = 160000
B = 64
DIN = 128
H1 = 64
H2 = 128
NH = 4
K = 10
ALPHA = 0.1

NP = 10112
DUMMY = 10008
EP = 170240
EPC = EP // 2
EPT = EPC // 16
KA = 760
STEPS_A = EPT // KA
EPT_B = EP // 16
KB = 560
STEPS_B = EPT_B // KB
RPT = NP // 16

_MESH = plsc.VectorSubcoreMesh(core_axis_name="c", subcore_axis_name="s")
_SC_PARAMS = pltpu.CompilerParams(use_tc_tiling_on_sc=False)
_IOTA = lambda: jax.lax.iota(jnp.int32, 16)
_ZCHUNKS = ((0, 280), (280, 280), (560, RPT - 560))


def _gat_a_body(as_hbm, ad_hbm, srcp_hbm, dstp_hbm, z16_hbm,
                ex_hbm, spart_hbm,
                sidx_v, didx_v, asr_v, adr_v, srow_v, stg_v, ssh, sem):
    c = lax.axis_index("c")
    s = lax.axis_index("s")
    for off, size in _ZCHUNKS:
        pltpu.sync_copy(z16_hbm.at[pl.ds(off, size)],
                        ssh.at[pl.ds(s * RPT + off, size)])
    plsc.subcore_barrier()
    base = c * EPC + s * EPT

    def step(i, carry):
        off = base + i * KA
        pltpu.sync_copy(srcp_hbm.at[pl.ds(off, KA)], sidx_v)
        pltpu.sync_copy(dstp_hbm.at[pl.ds(off, KA)], didx_v)
        pltpu.async_copy(as_hbm.at[sidx_v], asr_v, sem).wait()
        pltpu.async_copy(ad_hbm.at[didx_v], adr_v, sem).wait()

        def edge(e, carry2):
            v = asr_v[e] + adr_v[e]
            v = jnp.where(v > 0, v, 0.2 * v)
            v = jnp.exp(v)
            v = jnp.where(_IOTA() < 5, v, 0.0)
            srow_v[e] = v
            return carry2

        lax.fori_loop(0, KA, edge, 0)
        pltpu.sync_copy(srow_v, ssh.at[didx_v], add=True)
        pltpu.sync_copy(srow_v, ex_hbm.at[pl.ds(off, KA)])
        return carry

    lax.fori_loop(0, STEPS_A, step, 0)
    plsc.subcore_barrier()
    for off, size in _ZCHUNKS:
        pltpu.sync_copy(ssh.at[pl.ds(s * RPT + off, size)],
                        stg_v.at[pl.ds(0, size)])
        pltpu.sync_copy(stg_v.at[pl.ds(0, size)],
                        spart_hbm.at[pl.ds(c * NP + s * RPT + off, size)])


_gat_a = pl.kernel(
    _gat_a_body,
    out_type=(jax.ShapeDtypeStruct((EP, 16), jnp.float32),
              jax.ShapeDtypeStruct((2 * NP, 16), jnp.float32)),
    scratch_types=[
        pltpu.VMEM((KA,), jnp.int32),
        pltpu.VMEM((KA,), jnp.int32),
        pltpu.VMEM((KA, 16), jnp.float32),
        pltpu.VMEM((KA, 16), jnp.float32),
        pltpu.VMEM((KA, 16), jnp.float32),
        pltpu.VMEM((RPT, 16), jnp.float32),
        pltpu.VMEM_SHARED((NP, 16), jnp.float32),
        pltpu.SemaphoreType.DMA,
    ],
    mesh=_MESH,
    compiler_params=_SC_PARAMS,
)


def _gat_b_body(hw4_hbm, ex_hbm, srcp_hbm, dstp_hbm, z64_hbm,
                out_hbm,
                sidx_v, didx_v, sidx4_v, rows_v, stg_v, exv_v, acc_sh, sem):
    c = lax.axis_index("c")
    s = lax.axis_index("s")
    base = s * EPT_B
    for q in range(2):
        head_off = (2 * c + q) * NP
        for off, size in _ZCHUNKS:
            pltpu.sync_copy(z64_hbm.at[pl.ds(off, size)],
                            acc_sh.at[pl.ds(s * RPT + off, size)])
        plsc.subcore_barrier()

        def step(i, carry):
            off = base + i * KB
            pltpu.sync_copy(srcp_hbm.at[pl.ds(off, KB)], sidx_v)
            pltpu.sync_copy(dstp_hbm.at[pl.ds(off, KB)], didx_v)

            def shiftm(m, carry2):
                sidx4_v[pl.ds(16 * m, 16)] = (sidx_v[pl.ds(16 * m, 16)]
                                              + head_off)
                return carry2

            lax.fori_loop(0, KB // 16, shiftm, 0)
            pltpu.async_copy(hw4_hbm.at[sidx4_v], rows_v, sem).wait()
            pltpu.sync_copy(ex_hbm.at[pl.ds(off, KB)], exv_v)

            def edge(e, carry2):
                ex_row = exv_v[e]
                bq = jnp.where(c == 0,
                               jnp.full((16,), ex_row[q], jnp.float32),
                               jnp.full((16,), ex_row[2 + q], jnp.float32))
                for j in range(4):
                    sl = pl.ds(16 * j, 16)
                    rows_v[e, sl] = rows_v[e, sl] * bq
                return carry2

            lax.fori_loop(0, KB, edge, 0)
            pltpu.sync_copy(rows_v, acc_sh.at[didx_v], add=True)
            return carry

        lax.fori_loop(0, STEPS_B, step, 0)
        plsc.subcore_barrier()
        for off, size in _ZCHUNKS:
            pltpu.sync_copy(acc_sh.at[pl.ds(s * RPT + off, size)],
                            stg_v.at[pl.ds(0, size)])
            pltpu.sync_copy(stg_v.at[pl.ds(0, size)],
                            out_hbm.at[pl.ds(head_off + s * RPT + off, size)])
        plsc.subcore_barrier()


_gat_b = pl.kernel(
    _gat_b_body,
    out_type=jax.ShapeDtypeStruct((4 * NP, H1), jnp.float32),
    scratch_types=[
        pltpu.VMEM((KB,), jnp.int32),
        pltpu.VMEM((KB,), jnp.int32),
        pltpu.VMEM((KB,), jnp.int32),
        pltpu.VMEM((KB, H1), jnp.float32),
        pltpu.VMEM((RPT, H1), jnp.float32),
        pltpu.VMEM((KB, 16), jnp.float32),
        pltpu.VMEM_SHARED((NP, H1), jnp.float32),
        pltpu.SemaphoreType.DMA,
    ],
    mesh=_MESH,
    compiler_params=_SC_PARAMS,
)


def _appnp_sc_body(hp2_hbm, srcp_hbm, dstp_hbm, z64_hbm, out_hbm,
                   sidx_v, sidx2_v, didx_v, rows_v, stg_v, acc_sh, sem):
    c = lax.axis_index("c")
    s = lax.axis_index("s")
    for off, size in _ZCHUNKS:
        pltpu.sync_copy(z64_hbm.at[pl.ds(off, size)],
                        acc_sh.at[pl.ds(s * RPT + off, size)])
    plsc.subcore_barrier()
    base = s * EPT_B
    cNP = c * NP

    def step(i, carry):
        off = base + i * KB
        pltpu.sync_copy(srcp_hbm.at[pl.ds(off, KB)], sidx_v)
        pltpu.sync_copy(dstp_hbm.at[pl.ds(off, KB)], didx_v)

        def shiftm(m, carry2):
            sidx2_v[pl.ds(16 * m, 16)] = sidx_v[pl.ds(16 * m, 16)] + cNP
            return carry2

        lax.fori_loop(0, KB // 16, shiftm, 0)
        pltpu.async_copy(hp2_hbm.at[sidx2_v], rows_v, sem).wait()
        pltpu.sync_copy(rows_v, acc_sh.at[didx_v], add=True)
        return carry

    lax.fori_loop(0, STEPS_B, step, 0)
    plsc.subcore_barrier()
    for off, size in _ZCHUNKS:
        pltpu.sync_copy(acc_sh.at[pl.ds(s * RPT + off, size)],
                        stg_v.at[pl.ds(0, size)])
        pltpu.sync_copy(stg_v.at[pl.ds(0, size)],
                        out_hbm.at[pl.ds(cNP + s * RPT + off, size)])


_appnp_sc = pl.kernel(
    _appnp_sc_body,
    out_type=jax.ShapeDtypeStruct((2 * NP, H1), jnp.float32),
    scratch_types=[
        pltpu.VMEM((KB,), jnp.int32),
        pltpu.VMEM((KB,), jnp.int32),
        pltpu.VMEM((KB,), jnp.int32),
        pltpu.VMEM((KB, H1), jnp.float32),
        pltpu.VMEM((RPT, H1), jnp.float32),
        pltpu.VMEM_SHARED((NP, H1), jnp.float32),
        pltpu.SemaphoreType.DMA,
    ],
    mesh=_MESH,
    compiler_params=_SC_PARAMS,
)


def _combine_mid_body(out2_ref, x0_ref, dinv_ref, o_ref):
    ssum = jnp.concatenate([out2_ref[0:NP, :], out2_ref[NP:2 * NP, :]], 1)
    dinv = dinv_ref[...]
    h = dinv * ((1.0 - ALPHA) * dinv * ssum + ALPHA * x0_ref[...])
    o_ref[0:NP, :] = h[:, :H1]
    o_ref[NP:2 * NP, :] = h[:, H1:]


def _combine_last_body(out2_ref, x0_ref, dinv_ref, o_ref):
    ssum = jnp.concatenate([out2_ref[0:NP, :], out2_ref[NP:2 * NP, :]], 1)
    o_ref[...] = (1.0 - ALPHA) * dinv_ref[...] * ssum + ALPHA * x0_ref[...]


def _combine(out2, x0p, dinvp, last):
    if last:
        return pl.pallas_call(
            _combine_last_body,
            out_shape=jax.ShapeDtypeStruct((NP, H2), jnp.float32),
        )(out2, x0p, dinvp)
    return pl.pallas_call(
        _combine_mid_body,
        out_shape=jax.ShapeDtypeStruct((2 * NP, H1), jnp.float32),
    )(out2, x0p, dinvp)


def _lin1_body(x_ref, w_ref, b_ref, o_ref):
    o_ref[...] = jnp.dot(x_ref[...], w_ref[...],
                         preferred_element_type=jnp.float32) + b_ref[...]


def _lin1(xcat, w, b):
    return pl.pallas_call(
        _lin1_body,
        out_shape=jax.ShapeDtypeStruct((N, H2), jnp.float32),
    )(xcat, w, b[None, :])


def _bn(x, g, b):
    mu = jnp.mean(x, 0)
    var = jnp.var(x, 0)
    return (x - mu) / jnp.sqrt(var + 1e-5) * g + b


def _pad_rows(a):
    return jnp.pad(a, ((0, NP - a.shape[0]), (0, 0)))


def _gat_sc(h, srcp, dstp, p, z16, z64):
    hw = h @ p['gat_w']
    hwr = hw.reshape(N, NH, H1)
    a_s = jnp.sum(hwr * p['a_src'][None], -1)
    a_d = jnp.sum(hwr * p['a_dst'][None], -1)
    z12 = jnp.zeros((N, 12), jnp.float32)
    as_t = _pad_rows(jnp.concatenate([a_s, z12], 1))
    ad_t = _pad_rows(jnp.concatenate([a_d, z12], 1))
    ex, spart = _gat_a(as_t, ad_t, srcp, dstp, z16)
    scomb = spart[:NP] + spart[NP:]
    hw4 = jnp.concatenate(
        [_pad_rows(hw[:, i * H1:(i + 1) * H1]) for i in range(NH)], 0)
    outB = _gat_b(hw4, ex, srcp, dstp, z64)
    s_n = scomb[:N, 0:4] + 1e-16
    osum = sum(outB[i * NP:i * NP + N] / s_n[:, i:i + 1] for i in range(NH))
    return osum, scomb


def _att_layer_sc(x, srcp, dstp, p, z16, z64):
    osum, scomb = _gat_sc(x, srcp, dstp, p, z16, z64)
    h = osum @ p['lin1_w']
    h = _bn(x + h, p['bn1_g'], p['bn1_b'])
    h2 = jax.nn.relu(h @ p['lin2_w'] + p['lin2_b'])
    h2 = h2 @ p['lin3_w'] + p['lin3_b']
    return _bn(h2 + h, p['bn2_g'], p['bn2_b']), scomb


def _appnp(x, deg, srcp, dstp, z64):
    dinv = jnp.where(deg > 0, 1.0 / jnp.sqrt(deg), 0.0)
    dinvp = _pad_rows(dinv[:, None])
    x0p = _pad_rows(x)
    hp = dinvp * x0p
    hp2 = jnp.concatenate([hp[:, :H1], hp[:, H1:]], 0)
    for t in range(K):
        out2 = _appnp_sc(hp2, srcp, dstp, z64)
        hp2 = _combine(out2, x0p, dinvp, last=(t == K - 1))
    return hp2[:N]


def _pool(xg, onehot, p):
    gate = (jax.nn.relu(xg @ p['g1_w'] + p['g1_b']) @ p['g2_w'] + p['g2_b'])[:, 0]
    hh = jax.nn.relu(xg @ p['n1_w'] + p['n1_b']) @ p['n2_w'] + p['n2_b']
    ex = jnp.exp(gate)
    ssum = onehot.T @ ex
    a = (ex / (onehot @ ssum + 1e-16))[:, None]
    return onehot.T @ (a * hh)


def kernel(x, edge_index, batch, Omegas, Phis, Lambdas, J, saved_nodes,
           infected_nodes, size_connected, params):
    n = x.shape[0]
    b = Omegas.shape[0]
    loop = jnp.arange(n, dtype=edge_index.dtype)
    pad = jnp.full((EP - E - N,), DUMMY, edge_index.dtype)
    srcp = jnp.concatenate([edge_index[0], loop, pad])
    dstp = jnp.concatenate([edge_index[1], loop, pad])
    z16 = jnp.zeros((NP, 16), jnp.float32)
    z64 = jnp.zeros((NP, H1), jnp.float32)
    onehot = (batch[:, None] == jnp.arange(b)[None, :]).astype(jnp.float32)

    xcat = jnp.concatenate([x, J, size_connected], 1)
    h = _lin1(xcat, params['lin1_w'], params['lin1_b'])
    deg = None
    for i in (1, 2, 3):
        h, scomb = _att_layer_sc(h, srcp, dstp, params['att%d' % i], z16, z64)
        if deg is None:
            deg = scomb[:N, 4]
    h = _appnp(h, deg, srcp, dstp, z64)

    xg = jnp.concatenate([h, J, saved_nodes, infected_nodes], 1)
    g = jnp.concatenate([_pool(xg, onehot, params['pool%d' % j]) for j in (1, 2, 3)], 1)
    xs = jnp.concatenate([h, onehot @ g, J, saved_nodes, infected_nodes,
                          size_connected, onehot @ Omegas, onehot @ Phis,
                          onehot @ Lambdas], 1)
    s = xs @ params['lin3_w'] + params['lin3_b']
    s = jax.nn.leaky_relu(s, 0.2)
    s = _bn(s, params['bn1_g'], params['bn1_b'])
    s = s @ params['lin4_w'] + params['lin4_b']
    s = jax.nn.leaky_relu(s, 0.2)
    s = _bn(s, params['bn2_g'], params['bn2_b'])
    s = jax.nn.sigmoid(s @ params['lin5_w'] + params['lin5_b'])
    return onehot.T @ s


# speedup vs baseline: 1.5501x; 1.0918x over previous
import functools

import jax
import jax.numpy as jnp
from jax import lax
from jax.experimental import pallas as pl
from jax.experimental.pallas import tpu as pltpu
from jax.experimental.pallas import tpu_sc as plsc

N = 10000
E = 160000
B = 64
DIN = 128
H1 = 64
H2 = 128
NH = 4
K = 10
ALPHA = 0.1

NP = 10112
DUMMY = 10008
EP = 170240
EPC = EP // 2
EPT = EPC // 16
KA = 760
STEPS_A = EPT // KA
EPT_B = EP // 16
KB = 560
STEPS_B = EPT_B // KB
RPT = NP // 16

_MESH = plsc.VectorSubcoreMesh(core_axis_name="c", subcore_axis_name="s")
_SC_PARAMS = pltpu.CompilerParams(use_tc_tiling_on_sc=False)
_IOTA = lambda: jax.lax.iota(jnp.int32, 16)
_ZCHUNKS = ((0, 280), (280, 280), (560, RPT - 560))


def _gat_a_body(as_hbm, ad_hbm, srcp_hbm, dstp_hbm, z16_hbm,
                ex_hbm, spart_hbm,
                sidx_v, didx_v, asr_v, adr_v, srow_v, stg_v, ssh, sem):
    c = lax.axis_index("c")
    s = lax.axis_index("s")
    for off, size in _ZCHUNKS:
        pltpu.sync_copy(z16_hbm.at[pl.ds(off, size)],
                        ssh.at[pl.ds(s * RPT + off, size)])
    plsc.subcore_barrier()
    base = c * EPC + s * EPT

    def step(i, carry):
        off = base + i * KA
        pltpu.sync_copy(srcp_hbm.at[pl.ds(off, KA)], sidx_v)
        pltpu.sync_copy(dstp_hbm.at[pl.ds(off, KA)], didx_v)
        pltpu.async_copy(as_hbm.at[sidx_v], asr_v, sem).wait()
        pltpu.async_copy(ad_hbm.at[didx_v], adr_v, sem).wait()

        def edge(e, carry2):
            v = asr_v[e] + adr_v[e]
            v = jnp.where(v > 0, v, 0.2 * v)
            v = jnp.exp(v)
            v = jnp.where(_IOTA() < 5, v, 0.0)
            srow_v[e] = v
            return carry2

        lax.fori_loop(0, KA, edge, 0)
        pltpu.sync_copy(srow_v, ssh.at[didx_v], add=True)
        pltpu.sync_copy(srow_v, ex_hbm.at[pl.ds(off, KA)])
        return carry

    lax.fori_loop(0, STEPS_A, step, 0)
    plsc.subcore_barrier()
    for off, size in _ZCHUNKS:
        pltpu.sync_copy(ssh.at[pl.ds(s * RPT + off, size)],
                        stg_v.at[pl.ds(0, size)])
        pltpu.sync_copy(stg_v.at[pl.ds(0, size)],
                        spart_hbm.at[pl.ds(c * NP + s * RPT + off, size)])


_gat_a = pl.kernel(
    _gat_a_body,
    out_type=(jax.ShapeDtypeStruct((EP, 16), jnp.float32),
              jax.ShapeDtypeStruct((2 * NP, 16), jnp.float32)),
    scratch_types=[
        pltpu.VMEM((KA,), jnp.int32),
        pltpu.VMEM((KA,), jnp.int32),
        pltpu.VMEM((KA, 16), jnp.float32),
        pltpu.VMEM((KA, 16), jnp.float32),
        pltpu.VMEM((KA, 16), jnp.float32),
        pltpu.VMEM((RPT, 16), jnp.float32),
        pltpu.VMEM_SHARED((NP, 16), jnp.float32),
        pltpu.SemaphoreType.DMA,
    ],
    mesh=_MESH,
    compiler_params=_SC_PARAMS,
)


def _gat_b_body(hw4_hbm, ex_hbm, srcp_hbm, dstp_hbm, z64_hbm,
                out_hbm,
                sidx_v, didx_v, sidx4_v, rows_v, stg_v, exv_v, acc_sh, sem):
    c = lax.axis_index("c")
    s = lax.axis_index("s")
    base = s * EPT_B
    for q in range(2):
        head_off = (2 * c + q) * NP
        for off, size in _ZCHUNKS:
            pltpu.sync_copy(z64_hbm.at[pl.ds(off, size)],
                            acc_sh.at[pl.ds(s * RPT + off, size)])
        plsc.subcore_barrier()

        def step(i, carry):
            off = base + i * KB
            pltpu.sync_copy(srcp_hbm.at[pl.ds(off, KB)], sidx_v)
            pltpu.sync_copy(dstp_hbm.at[pl.ds(off, KB)], didx_v)

            def shiftm(m, carry2):
                sidx4_v[pl.ds(16 * m, 16)] = (sidx_v[pl.ds(16 * m, 16)]
                                              + head_off)
                return carry2

            lax.fori_loop(0, KB // 16, shiftm, 0)
            pltpu.async_copy(hw4_hbm.at[sidx4_v], rows_v, sem).wait()
            pltpu.sync_copy(ex_hbm.at[pl.ds(off, KB)], exv_v)

            def edge(e, carry2):
                ex_row = exv_v[e]
                bq = jnp.where(c == 0,
                               jnp.full((16,), ex_row[q], jnp.float32),
                               jnp.full((16,), ex_row[2 + q], jnp.float32))
                for j in range(4):
                    sl = pl.ds(16 * j, 16)
                    rows_v[e, sl] = rows_v[e, sl] * bq
                return carry2

            lax.fori_loop(0, KB, edge, 0)
            pltpu.sync_copy(rows_v, acc_sh.at[didx_v], add=True)
            return carry

        lax.fori_loop(0, STEPS_B, step, 0)
        plsc.subcore_barrier()
        for off, size in _ZCHUNKS:
            pltpu.sync_copy(acc_sh.at[pl.ds(s * RPT + off, size)],
                            stg_v.at[pl.ds(0, size)])
            pltpu.sync_copy(stg_v.at[pl.ds(0, size)],
                            out_hbm.at[pl.ds(head_off + s * RPT + off, size)])
        plsc.subcore_barrier()


_gat_b = pl.kernel(
    _gat_b_body,
    out_type=jax.ShapeDtypeStruct((4 * NP, H1), jnp.float32),
    scratch_types=[
        pltpu.VMEM((KB,), jnp.int32),
        pltpu.VMEM((KB,), jnp.int32),
        pltpu.VMEM((KB,), jnp.int32),
        pltpu.VMEM((KB, H1), jnp.float32),
        pltpu.VMEM((RPT, H1), jnp.float32),
        pltpu.VMEM((KB, 16), jnp.float32),
        pltpu.VMEM_SHARED((NP, H1), jnp.float32),
        pltpu.SemaphoreType.DMA,
    ],
    mesh=_MESH,
    compiler_params=_SC_PARAMS,
)


def _appnp_sc_body(hp2_hbm, srcp_hbm, dstp_hbm, z64_hbm, out_hbm,
                   sidxA, sidx2A, didxA, rowsA,
                   sidxB, sidx2B, didxB, rowsB, acc_sh, semA, semB):
    c = lax.axis_index("c")
    s = lax.axis_index("s")
    for off, size in _ZCHUNKS:
        pltpu.sync_copy(z64_hbm.at[pl.ds(off, size)],
                        acc_sh.at[pl.ds(s * RPT + off, size)])
    plsc.subcore_barrier()
    base = s * EPT_B
    cNP = c * NP

    def stage(i, sidx, sidx2, didx):
        off = base + i * KB
        pltpu.sync_copy(srcp_hbm.at[pl.ds(off, KB)], sidx)
        pltpu.sync_copy(dstp_hbm.at[pl.ds(off, KB)], didx)

        def shiftm(m, carry2):
            sidx2[pl.ds(16 * m, 16)] = sidx[pl.ds(16 * m, 16)] + cNP
            return carry2

        lax.fori_loop(0, KB // 16, shiftm, 0)

    stage(0, sidxA, sidx2A, didxA)
    pltpu.make_async_copy(hp2_hbm.at[sidx2A], rowsA, semA).start()

    def pair(p, carry):
        stage(2 * p + 1, sidxB, sidx2B, didxB)
        pltpu.make_async_copy(hp2_hbm.at[sidx2B], rowsB, semB).start()
        pltpu.make_async_copy(hp2_hbm.at[sidx2A], rowsA, semA).wait()
        pltpu.sync_copy(rowsA, acc_sh.at[didxA], add=True)
        stage(2 * p + 2, sidxA, sidx2A, didxA)
        pltpu.make_async_copy(hp2_hbm.at[sidx2A], rowsA, semA).start()
        pltpu.make_async_copy(hp2_hbm.at[sidx2B], rowsB, semB).wait()
        pltpu.sync_copy(rowsB, acc_sh.at[didxB], add=True)
        return carry

    lax.fori_loop(0, (STEPS_B - 1) // 2, pair, 0)
    pltpu.make_async_copy(hp2_hbm.at[sidx2A], rowsA, semA).wait()
    pltpu.sync_copy(rowsA, acc_sh.at[didxA], add=True)
    plsc.subcore_barrier()
    for off, size in _ZCHUNKS:
        pltpu.sync_copy(acc_sh.at[pl.ds(s * RPT + off, size)],
                        rowsA.at[pl.ds(0, size)])
        pltpu.sync_copy(rowsA.at[pl.ds(0, size)],
                        out_hbm.at[pl.ds(cNP + s * RPT + off, size)])


_appnp_sc = pl.kernel(
    _appnp_sc_body,
    out_type=jax.ShapeDtypeStruct((2 * NP, H1), jnp.float32),
    scratch_types=[
        pltpu.VMEM((KB,), jnp.int32),
        pltpu.VMEM((KB,), jnp.int32),
        pltpu.VMEM((KB,), jnp.int32),
        pltpu.VMEM((KB, H1), jnp.float32),
        pltpu.VMEM((KB,), jnp.int32),
        pltpu.VMEM((KB,), jnp.int32),
        pltpu.VMEM((KB,), jnp.int32),
        pltpu.VMEM((KB, H1), jnp.float32),
        pltpu.VMEM_SHARED((NP, H1), jnp.float32),
        pltpu.SemaphoreType.DMA,
        pltpu.SemaphoreType.DMA,
    ],
    mesh=_MESH,
    compiler_params=_SC_PARAMS,
)


def _combine_mid_body(out2_ref, x0_ref, dinv_ref, o_ref):
    ssum = jnp.concatenate([out2_ref[0:NP, :], out2_ref[NP:2 * NP, :]], 1)
    dinv = dinv_ref[...]
    h = dinv * ((1.0 - ALPHA) * dinv * ssum + ALPHA * x0_ref[...])
    o_ref[0:NP, :] = h[:, :H1]
    o_ref[NP:2 * NP, :] = h[:, H1:]


def _combine_last_body(out2_ref, x0_ref, dinv_ref, o_ref):
    ssum = jnp.concatenate([out2_ref[0:NP, :], out2_ref[NP:2 * NP, :]], 1)
    o_ref[...] = (1.0 - ALPHA) * dinv_ref[...] * ssum + ALPHA * x0_ref[...]


def _combine(out2, x0p, dinvp, last):
    if last:
        return pl.pallas_call(
            _combine_last_body,
            out_shape=jax.ShapeDtypeStruct((NP, H2), jnp.float32),
        )(out2, x0p, dinvp)
    return pl.pallas_call(
        _combine_mid_body,
        out_shape=jax.ShapeDtypeStruct((2 * NP, H1), jnp.float32),
    )(out2, x0p, dinvp)


def _lin1_body(x_ref, w_ref, b_ref, o_ref):
    o_ref[...] = jnp.dot(x_ref[...], w_ref[...],
                         preferred_element_type=jnp.float32) + b_ref[...]


def _lin1(xcat, w, b):
    return pl.pallas_call(
        _lin1_body,
        out_shape=jax.ShapeDtypeStruct((N, H2), jnp.float32),
    )(xcat, w, b[None, :])


def _bn(x, g, b):
    mu = jnp.mean(x, 0)
    var = jnp.var(x, 0)
    return (x - mu) / jnp.sqrt(var + 1e-5) * g + b


def _pad_rows(a):
    return jnp.pad(a, ((0, NP - a.shape[0]), (0, 0)))


def _gat_sc(h, srcp, dstp, p, z16, z64):
    hw = h @ p['gat_w']
    hwr = hw.reshape(N, NH, H1)
    a_s = jnp.sum(hwr * p['a_src'][None], -1)
    a_d = jnp.sum(hwr * p['a_dst'][None], -1)
    z12 = jnp.zeros((N, 12), jnp.float32)
    as_t = _pad_rows(jnp.concatenate([a_s, z12], 1))
    ad_t = _pad_rows(jnp.concatenate([a_d, z12], 1))
    ex, spart = _gat_a(as_t, ad_t, srcp, dstp, z16)
    scomb = spart[:NP] + spart[NP:]
    hw4 = jnp.concatenate(
        [_pad_rows(hw[:, i * H1:(i + 1) * H1]) for i in range(NH)], 0)
    outB = _gat_b(hw4, ex, srcp, dstp, z64)
    s_n = scomb[:N, 0:4] + 1e-16
    osum = sum(outB[i * NP:i * NP + N] / s_n[:, i:i + 1] for i in range(NH))
    return osum, scomb


def _att_layer_sc(x, srcp, dstp, p, z16, z64):
    osum, scomb = _gat_sc(x, srcp, dstp, p, z16, z64)
    h = osum @ p['lin1_w']
    h = _bn(x + h, p['bn1_g'], p['bn1_b'])
    h2 = jax.nn.relu(h @ p['lin2_w'] + p['lin2_b'])
    h2 = h2 @ p['lin3_w'] + p['lin3_b']
    return _bn(h2 + h, p['bn2_g'], p['bn2_b']), scomb


def _appnp(x, deg, srcp, dstp, z64):
    dinv = jnp.where(deg > 0, 1.0 / jnp.sqrt(deg), 0.0)
    dinvp = _pad_rows(dinv[:, None])
    x0p = _pad_rows(x)
    hp = dinvp * x0p
    hp2 = jnp.concatenate([hp[:, :H1], hp[:, H1:]], 0)
    for t in range(K):
        out2 = _appnp_sc(hp2, srcp, dstp, z64)
        hp2 = _combine(out2, x0p, dinvp, last=(t == K - 1))
    return hp2[:N]


def _pool(xg, onehot, p):
    gate = (jax.nn.relu(xg @ p['g1_w'] + p['g1_b']) @ p['g2_w'] + p['g2_b'])[:, 0]
    hh = jax.nn.relu(xg @ p['n1_w'] + p['n1_b']) @ p['n2_w'] + p['n2_b']
    ex = jnp.exp(gate)
    ssum = onehot.T @ ex
    a = (ex / (onehot @ ssum + 1e-16))[:, None]
    return onehot.T @ (a * hh)


def kernel(x, edge_index, batch, Omegas, Phis, Lambdas, J, saved_nodes,
           infected_nodes, size_connected, params):
    n = x.shape[0]
    b = Omegas.shape[0]
    loop = jnp.arange(n, dtype=edge_index.dtype)
    pad = jnp.full((EP - E - N,), DUMMY, edge_index.dtype)
    srcp = jnp.concatenate([edge_index[0], loop, pad])
    dstp = jnp.concatenate([edge_index[1], loop, pad])
    z16 = jnp.zeros((NP, 16), jnp.float32)
    z64 = jnp.zeros((NP, H1), jnp.float32)
    onehot = (batch[:, None] == jnp.arange(b)[None, :]).astype(jnp.float32)

    xcat = jnp.concatenate([x, J, size_connected], 1)
    h = _lin1(xcat, params['lin1_w'], params['lin1_b'])
    deg = None
    for i in (1, 2, 3):
        h, scomb = _att_layer_sc(h, srcp, dstp, params['att%d' % i], z16, z64)
        if deg is None:
            deg = scomb[:N, 4]
    h = _appnp(h, deg, srcp, dstp, z64)

    xg = jnp.concatenate([h, J, saved_nodes, infected_nodes], 1)
    g = jnp.concatenate([_pool(xg, onehot, params['pool%d' % j]) for j in (1, 2, 3)], 1)
    xs = jnp.concatenate([h, onehot @ g, J, saved_nodes, infected_nodes,
                          size_connected, onehot @ Omegas, onehot @ Phis,
                          onehot @ Lambdas], 1)
    s = xs @ params['lin3_w'] + params['lin3_b']
    s = jax.nn.leaky_relu(s, 0.2)
    s = _bn(s, params['bn1_g'], params['bn1_b'])
    s = s @ params['lin4_w'] + params['lin4_b']
    s = jax.nn.leaky_relu(s, 0.2)
    s = _bn(s, params['bn2_g'], params['bn2_b'])
    s = jax.nn.sigmoid(s @ params['lin5_w'] + params['lin5_b'])
    return onehot.T @ s
